# baseline (speedup 1.0000x reference)
import jax
import jax.numpy as jnp
from jax.experimental import pallas as pl
from jax.experimental.pallas import tpu as pltpu

_NEI = 8
_K = 16
_D = 64
_C = 1920
_N = 50000


def _io(shape, dim):
    return jax.lax.broadcasted_iota(jnp.int32, shape, dim)


def _weights_and_selectors(wq, wk, w):
    f32 = jnp.float32
    mt = jax.lax.dot_general(wk, wq, (((0,), (0,)), ((), ())),
                             preferred_element_type=f32)
    r, c = _io((_D, _D), 0), _io((_D, _D), 1)
    wt = jnp.where(r <= c, w, 0.0)
    eye = jnp.where(r == c, 1.0, 0.0).astype(f32)
    wt_t = jax.lax.dot_general(eye, wt, (((1,), (1,)), ((), ())),
                               preferred_element_type=f32)
    ws = wt + wt_t
    sum16 = (_io((_K, _K * _D), 1) // _D == _io((_K, _K * _D), 0)).astype(f32)
    spr3 = (_io((_K * 3, _K), 0) % _K == _io((_K * 3, _K), 1)).astype(f32)
    sum3 = (_io((3, _K * 3), 1) // _K == _io((3, _K * 3), 0)).astype(f32)
    bsp = (_io((_K * _K, _K), 0) // _K == _io((_K * _K, _K), 1)).astype(f32)
    asel = (_io((_K, _K * _K), 1) % _K == _io((_K, _K * _K), 0)).astype(f32)
    return mt, ws, sum16, spr3, sum3, bsp, asel


def _dot(a, b):
    return jnp.dot(a, b, preferred_element_type=jnp.float32)


def _rank_in_topk(s, bsp, asel):
    f32 = jnp.float32
    srep = pltpu.repeat(s, _K, axis=0)
    sspr = _dot(bsp, s)
    a_io = _io((_K * _K, 1), 0) // _K
    j_io = _io((_K * _K, 1), 0) % _K
    beats = (sspr > srep) | ((sspr == srep) & (a_io < j_io))
    rank = _dot(asel, beats.astype(f32))
    return rank < float(_NEI)


def _softmax_weighted(s, kf_ref, xyz_ref, spr3, sum3):
    mx = jnp.max(s, axis=0, keepdims=True)
    e = jnp.exp(s - mx)
    p = e / jnp.sum(e, axis=0, keepdims=True)
    cf = p[0:1, :] * kf_ref[0:_D, :]
    for j in range(1, _K):
        cf = cf + p[j:j + 1, :] * kf_ref[j * _D:(j + 1) * _D, :]
    p3 = _dot(spr3, p)
    cx = _dot(sum3, p3 * xyz_ref[...])
    return p, cf, cx


def _pass1(n_total, feat_ref, xyz_ref, kf_ref, wq_ref, wk_ref, w_ref,
           ml_ref, att_ref, cx_ref, un_ref, cst):
    f32 = jnp.float32
    i = pl.program_id(0)

    @pl.when(i == 0)
    def _init():
        vals = _weights_and_selectors(wq_ref[...], wk_ref[...], w_ref[...])
        for ref, v in zip(cst, vals):
            ref[...] = v
        un_ref[...] = jnp.zeros(un_ref.shape, f32)

    mt, ws, sum16, spr3, sum3, bsp, asel = (cst[j][...] for j in range(7))
    pq_scr, pw_scr = cst[7], cst[8]
    f = feat_ref[...]
    fq = _dot(mt, f)
    fw = _dot(ws, f)
    for j in range(_K):
        sl = pl.ds(j * _D, _D)
        kfj = kf_ref[sl, :]
        pq_scr[sl, :] = (fq * kfj).astype(jnp.bfloat16)
        pw_scr[sl, :] = (fw * kfj).astype(jnp.bfloat16)
    s = _dot(sum16.astype(jnp.bfloat16), pq_scr[...])
    ml = _dot(sum16.astype(jnp.bfloat16), pw_scr[...])
    ml_ref[...] = ml

    p, cf, cx = _softmax_weighted(s, kf_ref, xyz_ref, spr3, sum3)
    cx_ref[...] = cx
    att_ref[0:_D, :] = f
    att_ref[_D:2 * _D, :] = cf
    att_ref[2 * _D:2 * _D + 1, :] = jnp.sum(p * ml, axis=0, keepdims=True)

    in_top = _rank_in_topk(s, bsp, asel)
    valid = (i * _C + _io((1, _C), 1)) < n_total
    un_blk = jnp.max((in_top & valid).astype(f32), axis=1, keepdims=True)
    un_ref[...] = jnp.maximum(un_ref[...],
                              jnp.broadcast_to(un_blk, un_ref.shape))


def _pass2(mask_ref, feat_ref, xyz_ref, kf_ref, wq_ref, wk_ref, w_ref,
           att_ref, cx_ref, cst):
    i = pl.program_id(0)

    @pl.when(i == 0)
    def _init():
        vals = _weights_and_selectors(wq_ref[...], wk_ref[...], w_ref[...])
        for ref, v in zip(cst, vals):
            ref[...] = v

    mt, ws, sum16, spr3, sum3, bsp, asel = (cst[j][...] for j in range(7))
    pq_scr, pw_scr = cst[7], cst[8]
    f = feat_ref[...]
    fq = _dot(mt, f)
    fw = _dot(ws, f)
    for j in range(_K):
        sl = pl.ds(j * _D, _D)
        kfj = kf_ref[sl, :]
        pq_scr[sl, :] = (fq * kfj).astype(jnp.bfloat16)
        pw_scr[sl, :] = (fw * kfj).astype(jnp.bfloat16)
    s = _dot(sum16.astype(jnp.bfloat16), pq_scr[...]) + mask_ref[:, 0:1]
    ml = _dot(sum16.astype(jnp.bfloat16), pw_scr[...])
    p, cf, cx = _softmax_weighted(s, kf_ref, xyz_ref, spr3, sum3)
    cx_ref[...] = cx
    att_ref[0:_D, :] = f
    att_ref[_D:2 * _D, :] = cf
    att_ref[2 * _D:2 * _D + 1, :] = jnp.sum(p * ml, axis=0, keepdims=True)


def _const_scratch():
    f32 = jnp.float32
    return [pltpu.VMEM((_D, _D), f32), pltpu.VMEM((_D, _D), f32),
            pltpu.VMEM((_K, _K * _D), f32), pltpu.VMEM((_K * 3, _K), f32),
            pltpu.VMEM((3, _K * 3), f32), pltpu.VMEM((_K * _K, _K), f32),
            pltpu.VMEM((_K, _K * _K), f32),
            pltpu.VMEM((_K * _D, _C), jnp.bfloat16),
            pltpu.VMEM((_K * _D, _C), jnp.bfloat16)]


def kernel(feat, knn_xyz, knn_feat, Wq, Wk, W):
    f32 = jnp.float32
    n, d = feat.shape
    k = knn_feat.shape[1]
    assert d == _D and k == _K
    nb = (n + _C - 1) // _C

    ft = feat.T
    kft = knn_feat.transpose(1, 2, 0).reshape(k * d, n)
    xyzt = knn_xyz.transpose(2, 1, 0).reshape(k * 3, n)

    col_spec = lambda bs: pl.BlockSpec(bs, lambda i: (0, i))
    w_spec = pl.BlockSpec((d, d), lambda i: (0, 0))

    def _wrap1(*refs):
        _pass1(n, *refs[:10], refs[10:])

    ml, att0, cx0, un = pl.pallas_call(
        _wrap1,
        grid=(nb,),
        in_specs=[col_spec((d, _C)), col_spec((k * 3, _C)),
                  col_spec((k * d, _C)), w_spec, w_spec, w_spec],
        out_specs=[col_spec((k, _C)), col_spec((2 * d + 1, _C)),
                   col_spec((3, _C)),
                   pl.BlockSpec((k, 128), lambda i: (0, 0))],
        out_shape=[jax.ShapeDtypeStruct((k, n), f32),
                   jax.ShapeDtypeStruct((2 * d + 1, n), f32),
                   jax.ShapeDtypeStruct((3, n), f32),
                   jax.ShapeDtypeStruct((k, 128), f32)],
        scratch_shapes=_const_scratch(),
    )(ft, xyzt, kft, Wq, Wk, W)

    all_unmasked = jnp.all(un[:, 0] > 0.5)
    maskt = jnp.where(un > 0.5, 0.0, -jnp.inf).astype(f32)

    def _fast():
        return att0, cx0

    def _slow():
        def _wrap2(*refs):
            _pass2(*refs[:9], refs[9:])
        return pl.pallas_call(
            _wrap2,
            grid=(nb,),
            in_specs=[pl.BlockSpec((k, 128), lambda i: (0, 0)),
                      col_spec((d, _C)), col_spec((k * 3, _C)),
                      col_spec((k * d, _C)), w_spec, w_spec, w_spec],
            out_specs=[col_spec((2 * d + 1, _C)), col_spec((3, _C))],
            out_shape=[jax.ShapeDtypeStruct((2 * d + 1, n), f32),
                       jax.ShapeDtypeStruct((3, n), f32)],
            scratch_shapes=_const_scratch(),
        )(maskt, ft, xyzt, kft, Wq, Wk, W)

    att, cx = jax.lax.cond(all_unmasked, _fast, _slow)
    return (cx.T, att.T, ml.T)


# speedup vs baseline: 1.0010x; 1.0010x over previous
import jax
import jax.numpy as jnp
from jax.experimental import pallas as pl
from jax.experimental.pallas import tpu as pltpu

_NEI = 8
_K = 16
_D = 64
_C = 1920
_N = 50000


def _io(shape, dim):
    return jax.lax.broadcasted_iota(jnp.int32, shape, dim)


def _weights_and_selectors(wq, wk, w):
    f32 = jnp.float32
    mt = jax.lax.dot_general(wk, wq, (((0,), (0,)), ((), ())),
                             preferred_element_type=f32)
    r, c = _io((_D, _D), 0), _io((_D, _D), 1)
    wt = jnp.where(r <= c, w, 0.0)
    eye = jnp.where(r == c, 1.0, 0.0).astype(f32)
    wt_t = jax.lax.dot_general(eye, wt, (((1,), (1,)), ((), ())),
                               preferred_element_type=f32)
    ws = wt + wt_t
    sum16 = (_io((_K, _K * _D), 1) // _D == _io((_K, _K * _D), 0)).astype(f32)
    spr3 = (_io((_K * 3, _K), 0) % _K == _io((_K * 3, _K), 1)).astype(f32)
    sum3 = (_io((3, _K * 3), 1) // _K == _io((3, _K * 3), 0)).astype(f32)
    bsp = (_io((_K * _K, _K), 0) // _K == _io((_K * _K, _K), 1)).astype(f32)
    asel = (_io((_K, _K * _K), 1) % _K == _io((_K, _K * _K), 0)).astype(f32)
    return mt, ws, sum16, spr3, sum3, bsp, asel


def _dot(a, b):
    return jnp.dot(a, b, preferred_element_type=jnp.float32)


def _rank_in_topk(s, bsp, asel):
    f32 = jnp.float32
    srep = pltpu.repeat(s, _K, axis=0)
    sspr = _dot(bsp, s)
    a_io = _io((_K * _K, 1), 0) // _K
    j_io = _io((_K * _K, 1), 0) % _K
    beats = (sspr > srep) | ((sspr == srep) & (a_io < j_io))
    rank = _dot(asel, beats.astype(f32))
    return rank < float(_NEI)


def _softmax_weighted(s, kf_ref, xyz_ref, spr3, sum3):
    mx = jnp.max(s, axis=0, keepdims=True)
    e = jnp.exp(s - mx)
    p = e / jnp.sum(e, axis=0, keepdims=True)
    cf = p[0:1, :] * kf_ref[0:_D, :]
    for j in range(1, _K):
        cf = cf + p[j:j + 1, :] * kf_ref[j * _D:(j + 1) * _D, :]
    p3 = _dot(spr3, p)
    cx = _dot(sum3, p3 * xyz_ref[...])
    return p, cf, cx


def _pass1(n_total, feat_ref, xyz_ref, kf_ref, wq_ref, wk_ref, w_ref,
           ml_ref, att_ref, cx_ref, un_ref, cst):
    f32 = jnp.float32
    i = pl.program_id(0)

    @pl.when(i == 0)
    def _init():
        vals = _weights_and_selectors(wq_ref[...], wk_ref[...], w_ref[...])
        for ref, v in zip(cst, vals):
            ref[...] = v
        un_ref[...] = jnp.zeros(un_ref.shape, f32)

    mt, ws, sum16, spr3, sum3, bsp, asel = (cst[j][...] for j in range(7))
    pq_scr, pw_scr = cst[7], cst[8]
    f = feat_ref[...]
    fq = _dot(mt, f)
    fw = _dot(ws, f)
    for j in range(_K):
        sl = pl.ds(j * _D, _D)
        kfj = kf_ref[sl, :]
        pq_scr[sl, :] = fq * kfj
        pw_scr[sl, :] = (fw * kfj).astype(jnp.bfloat16)
    s = _dot(sum16, pq_scr[...])
    ml = _dot(sum16.astype(jnp.bfloat16), pw_scr[...])
    ml_ref[...] = ml

    p, cf, cx = _softmax_weighted(s, kf_ref, xyz_ref, spr3, sum3)
    cx_ref[...] = cx
    att_ref[0:_D, :] = f
    att_ref[_D:2 * _D, :] = cf
    att_ref[2 * _D:2 * _D + 1, :] = jnp.sum(p * ml, axis=0, keepdims=True)

    in_top = _rank_in_topk(s, bsp, asel)
    valid = (i * _C + _io((1, _C), 1)) < n_total
    un_blk = jnp.max((in_top & valid).astype(f32), axis=1, keepdims=True)
    un_ref[...] = jnp.maximum(un_ref[...],
                              jnp.broadcast_to(un_blk, un_ref.shape))


def _pass2(mask_ref, feat_ref, xyz_ref, kf_ref, wq_ref, wk_ref, w_ref,
           att_ref, cx_ref, cst):
    i = pl.program_id(0)

    @pl.when(i == 0)
    def _init():
        vals = _weights_and_selectors(wq_ref[...], wk_ref[...], w_ref[...])
        for ref, v in zip(cst, vals):
            ref[...] = v

    mt, ws, sum16, spr3, sum3, bsp, asel = (cst[j][...] for j in range(7))
    pq_scr, pw_scr = cst[7], cst[8]
    f = feat_ref[...]
    fq = _dot(mt, f)
    fw = _dot(ws, f)
    for j in range(_K):
        sl = pl.ds(j * _D, _D)
        kfj = kf_ref[sl, :]
        pq_scr[sl, :] = fq * kfj
        pw_scr[sl, :] = (fw * kfj).astype(jnp.bfloat16)
    s = _dot(sum16, pq_scr[...]) + mask_ref[:, 0:1]
    ml = _dot(sum16.astype(jnp.bfloat16), pw_scr[...])
    p, cf, cx = _softmax_weighted(s, kf_ref, xyz_ref, spr3, sum3)
    cx_ref[...] = cx
    att_ref[0:_D, :] = f
    att_ref[_D:2 * _D, :] = cf
    att_ref[2 * _D:2 * _D + 1, :] = jnp.sum(p * ml, axis=0, keepdims=True)


def _const_scratch():
    f32 = jnp.float32
    return [pltpu.VMEM((_D, _D), f32), pltpu.VMEM((_D, _D), f32),
            pltpu.VMEM((_K, _K * _D), f32), pltpu.VMEM((_K * 3, _K), f32),
            pltpu.VMEM((3, _K * 3), f32), pltpu.VMEM((_K * _K, _K), f32),
            pltpu.VMEM((_K, _K * _K), f32),
            pltpu.VMEM((_K * _D, _C), f32),
            pltpu.VMEM((_K * _D, _C), jnp.bfloat16)]


def kernel(feat, knn_xyz, knn_feat, Wq, Wk, W):
    f32 = jnp.float32
    n, d = feat.shape
    k = knn_feat.shape[1]
    assert d == _D and k == _K
    nb = (n + _C - 1) // _C

    ft = feat.T
    kft = knn_feat.transpose(1, 2, 0).reshape(k * d, n)
    xyzt = knn_xyz.transpose(2, 1, 0).reshape(k * 3, n)

    col_spec = lambda bs: pl.BlockSpec(bs, lambda i: (0, i))
    w_spec = pl.BlockSpec((d, d), lambda i: (0, 0))

    def _wrap1(*refs):
        _pass1(n, *refs[:10], refs[10:])

    ml, att0, cx0, un = pl.pallas_call(
        _wrap1,
        grid=(nb,),
        in_specs=[col_spec((d, _C)), col_spec((k * 3, _C)),
                  col_spec((k * d, _C)), w_spec, w_spec, w_spec],
        out_specs=[col_spec((k, _C)), col_spec((2 * d + 1, _C)),
                   col_spec((3, _C)),
                   pl.BlockSpec((k, 128), lambda i: (0, 0))],
        out_shape=[jax.ShapeDtypeStruct((k, n), f32),
                   jax.ShapeDtypeStruct((2 * d + 1, n), f32),
                   jax.ShapeDtypeStruct((3, n), f32),
                   jax.ShapeDtypeStruct((k, 128), f32)],
        scratch_shapes=_const_scratch(),
    )(ft, xyzt, kft, Wq, Wk, W)

    all_unmasked = jnp.all(un[:, 0] > 0.5)
    maskt = jnp.where(un > 0.5, 0.0, -jnp.inf).astype(f32)

    def _fast():
        return att0, cx0

    def _slow():
        def _wrap2(*refs):
            _pass2(*refs[:9], refs[9:])
        return pl.pallas_call(
            _wrap2,
            grid=(nb,),
            in_specs=[pl.BlockSpec((k, 128), lambda i: (0, 0)),
                      col_spec((d, _C)), col_spec((k * 3, _C)),
                      col_spec((k * d, _C)), w_spec, w_spec, w_spec],
            out_specs=[col_spec((2 * d + 1, _C)), col_spec((3, _C))],
            out_shape=[jax.ShapeDtypeStruct((2 * d + 1, n), f32),
                       jax.ShapeDtypeStruct((3, n), f32)],
            scratch_shapes=_const_scratch(),
        )(maskt, ft, xyzt, kft, Wq, Wk, W)

    att, cx = jax.lax.cond(all_unmasked, _fast, _slow)
    return (cx.T, att.T, ml.T)


# speedup vs baseline: 1.0020x; 1.0010x over previous
import jax
import jax.numpy as jnp
from jax.experimental import pallas as pl
from jax.experimental.pallas import tpu as pltpu

_NEI = 8
_K = 16
_D = 64
_C = 1920


def _io(shape, dim):
    return jax.lax.broadcasted_iota(jnp.int32, shape, dim)


def _weights_and_selectors(wq, wk, w):
    f32 = jnp.float32
    mt = jax.lax.dot_general(wk, wq, (((0,), (0,)), ((), ())),
                             preferred_element_type=f32)
    r, c = _io((_D, _D), 0), _io((_D, _D), 1)
    wt = jnp.where(r <= c, w, 0.0)
    eye = jnp.where(r == c, 1.0, 0.0).astype(f32)
    wt_t = jax.lax.dot_general(eye, wt, (((1,), (1,)), ((), ())),
                               preferred_element_type=f32)
    ws = wt + wt_t
    sum16 = (_io((_K, _K * _D), 1) // _D == _io((_K, _K * _D), 0)).astype(f32)
    spr3 = (_io((_K * 3, _K), 0) % _K == _io((_K * 3, _K), 1)).astype(f32)
    sum3 = (_io((3, _K * 3), 1) // _K == _io((3, _K * 3), 0)).astype(f32)
    bsp = (_io((_K * _K, _K), 0) // _K == _io((_K * _K, _K), 1)).astype(f32)
    asel = (_io((_K, _K * _K), 1) % _K == _io((_K, _K * _K), 0)).astype(f32)
    return mt, ws, sum16, spr3, sum3, bsp, asel


def _dot(a, b):
    return jnp.dot(a, b, preferred_element_type=jnp.float32)


def _rank_in_topk(s, bsp, asel):
    f32 = jnp.float32
    srep = pltpu.repeat(s, _K, axis=0)
    sspr = _dot(bsp, s)
    a_io = _io((_K * _K, 1), 0) // _K
    j_io = _io((_K * _K, 1), 0) % _K
    beats = (sspr > srep) | ((sspr == srep) & (a_io < j_io))
    rank = _dot(asel, beats.astype(f32))
    return rank < float(_NEI)


def _softmax_weighted(s, kf_ref, xyz_ref, spr3, sum3):
    mx = jnp.max(s, axis=0, keepdims=True)
    e = jnp.exp(s - mx)
    p = e / jnp.sum(e, axis=0, keepdims=True)
    cf = p[0:1, :] * kf_ref[0:_D, :]
    for j in range(1, _K):
        cf = cf + p[j:j + 1, :] * kf_ref[j * _D:(j + 1) * _D, :]
    p3 = _dot(spr3, p)
    cx = _dot(sum3, p3 * xyz_ref[...])
    return p, cf, cx


def _pass1(n_total, feat_ref, xyz_ref, kf_ref, wq_ref, wk_ref, w_ref,
           ml_ref, att_ref, cx_ref, un_ref, cst):
    f32 = jnp.float32
    i = pl.program_id(0)

    @pl.when(i == 0)
    def _init():
        vals = _weights_and_selectors(wq_ref[...], wk_ref[...], w_ref[...])
        for ref, v in zip(cst, vals):
            ref[...] = v
        un_ref[...] = jnp.zeros(un_ref.shape, f32)

    mt, ws, sum16, spr3, sum3, bsp, asel = (cst[j][...] for j in range(7))
    pq_scr, pw_scr = cst[7], cst[8]
    f = feat_ref[...]
    fq = _dot(mt, f)
    fw = _dot(ws, f)
    for j in range(_K):
        sl = pl.ds(j * _D, _D)
        kfj = kf_ref[sl, :]
        pq_scr[sl, :] = fq * kfj
        pw_scr[sl, :] = (fw * kfj).astype(jnp.bfloat16)
    s = _dot(sum16, pq_scr[...])
    ml = _dot(sum16.astype(jnp.bfloat16), pw_scr[...])
    ml_ref[...] = ml

    p, cf, cx = _softmax_weighted(s, kf_ref, xyz_ref, spr3, sum3)
    cx_ref[...] = cx
    att_ref[0:_D, :] = f
    att_ref[_D:2 * _D, :] = cf
    att_ref[2 * _D:2 * _D + 1, :] = jnp.sum(p * ml, axis=0, keepdims=True)

    in_top = _rank_in_topk(s, bsp, asel)
    valid = (i * _C + _io((1, _C), 1)) < n_total
    un_blk = jnp.max((in_top & valid).astype(f32), axis=1, keepdims=True)
    un_ref[...] = jnp.maximum(un_ref[...],
                              jnp.broadcast_to(un_blk, un_ref.shape))


def _pass2(mask_ref, feat_ref, xyz_ref, kf_ref, wq_ref, wk_ref, w_ref,
           att_ref, cx_ref, cst):
    i = pl.program_id(0)

    @pl.when(i == 0)
    def _init():
        vals = _weights_and_selectors(wq_ref[...], wk_ref[...], w_ref[...])
        for ref, v in zip(cst, vals):
            ref[...] = v

    mt, ws, sum16, spr3, sum3, bsp, asel = (cst[j][...] for j in range(7))
    pq_scr, pw_scr = cst[7], cst[8]
    f = feat_ref[...]
    fq = _dot(mt, f)
    fw = _dot(ws, f)
    for j in range(_K):
        sl = pl.ds(j * _D, _D)
        kfj = kf_ref[sl, :]
        pq_scr[sl, :] = fq * kfj
        pw_scr[sl, :] = (fw * kfj).astype(jnp.bfloat16)
    s = _dot(sum16, pq_scr[...]) + mask_ref[:, 0:1]
    ml = _dot(sum16.astype(jnp.bfloat16), pw_scr[...])
    p, cf, cx = _softmax_weighted(s, kf_ref, xyz_ref, spr3, sum3)
    cx_ref[...] = cx
    att_ref[0:_D, :] = f
    att_ref[_D:2 * _D, :] = cf
    att_ref[2 * _D:2 * _D + 1, :] = jnp.sum(p * ml, axis=0, keepdims=True)


def _const_scratch():
    f32 = jnp.float32
    return [pltpu.VMEM((_D, _D), f32), pltpu.VMEM((_D, _D), f32),
            pltpu.VMEM((_K, _K * _D), f32), pltpu.VMEM((_K * 3, _K), f32),
            pltpu.VMEM((3, _K * 3), f32), pltpu.VMEM((_K * _K, _K), f32),
            pltpu.VMEM((_K, _K * _K), f32),
            pltpu.VMEM((_K * _D, _C), f32),
            pltpu.VMEM((_K * _D, _C), jnp.bfloat16)]


def kernel(feat, knn_xyz, knn_feat, Wq, Wk, W):
    f32 = jnp.float32
    n, d = feat.shape
    k = knn_feat.shape[1]
    assert d == _D and k == _K
    nb = (n + _C - 1) // _C

    ft = feat.T
    kft = knn_feat.transpose(1, 2, 0).reshape(k * d, n)
    xyzt = knn_xyz.transpose(2, 1, 0).reshape(k * 3, n)

    col_spec = lambda bs: pl.BlockSpec(bs, lambda i: (0, i))
    w_spec = pl.BlockSpec((d, d), lambda i: (0, 0))

    def _wrap1(*refs):
        _pass1(n, *refs[:10], refs[10:])

    ml, att0, cx0, un = pl.pallas_call(
        _wrap1,
        grid=(nb,),
        in_specs=[col_spec((d, _C)), col_spec((k * 3, _C)),
                  col_spec((k * d, _C)), w_spec, w_spec, w_spec],
        out_specs=[col_spec((k, _C)), col_spec((2 * d + 1, _C)),
                   col_spec((3, _C)),
                   pl.BlockSpec((k, 128), lambda i: (0, 0))],
        out_shape=[jax.ShapeDtypeStruct((k, n), f32),
                   jax.ShapeDtypeStruct((2 * d + 1, n), f32),
                   jax.ShapeDtypeStruct((3, n), f32),
                   jax.ShapeDtypeStruct((k, 128), f32)],
        scratch_shapes=_const_scratch(),
    )(ft, xyzt, kft, Wq, Wk, W)

    all_unmasked = jnp.all(un[:, 0] > 0.5)
    maskt = jnp.where(un > 0.5, 0.0, -jnp.inf).astype(f32)

    def _fast():
        return att0, cx0

    def _slow():
        def _wrap2(*refs):
            _pass2(*refs[:9], refs[9:])
        return pl.pallas_call(
            _wrap2,
            grid=(nb,),
            in_specs=[pl.BlockSpec((k, 128), lambda i: (0, 0)),
                      col_spec((d, _C)), col_spec((k * 3, _C)),
                      col_spec((k * d, _C)), w_spec, w_spec, w_spec],
            out_specs=[col_spec((2 * d + 1, _C)), col_spec((3, _C))],
            out_shape=[jax.ShapeDtypeStruct((2 * d + 1, n), f32),
                       jax.ShapeDtypeStruct((3, n), f32)],
            scratch_shapes=_const_scratch(),
        )(maskt, ft, xyzt, kft, Wq, Wk, W)

    att, cx = jax.lax.cond(all_unmasked, _fast, _slow)
    return (cx.T, att.T, ml.T)
